# Initial kernel scaffold; baseline (speedup 1.0000x reference)
import jax
import jax.numpy as jnp
from jax.experimental import pallas as pl


def kernel(x, edge_index, edge_attr, W_atom, W_bond, Wn, We, node_rand, edge_rand):
    raise NotImplementedError("write your pallas kernel here")


# speedup vs baseline: 1.0309x; 1.0309x over previous
import functools

import jax
import jax.numpy as jnp
from jax.experimental import pallas as pl
from jax.experimental.pallas import tpu as pltpu

N = 10000
E = 320000
DIN = 128
DE = 16
D = 128
P_NODE = 0.1
P_EDGE = 0.1


def _embed_body(a_ref, w_ref, m_ref, o_ref):
    o_ref[...] = jnp.dot(a_ref[...], w_ref[...],
                         preferred_element_type=jnp.float32) * m_ref[...]


def _embed(a, w, m, bm):
    r, k = a.shape
    return pl.pallas_call(
        _embed_body,
        grid=(r // bm,),
        in_specs=[
            pl.BlockSpec((bm, k), lambda i: (i, 0)),
            pl.BlockSpec((k, D), lambda i: (0, 0)),
            pl.BlockSpec((bm, 1), lambda i: (i, 0)),
        ],
        out_specs=pl.BlockSpec((bm, D), lambda i: (i, 0)),
        out_shape=jax.ShapeDtypeStruct((r, D), jnp.float32),
    )(a, w, m)


def _eupd_body(e_ref, w_ref, o_ref):
    e = e_ref[...]
    o_ref[...] = jnp.maximum(
        jnp.dot(e, w_ref[...], preferred_element_type=jnp.float32), 0.0) + e


def _e_update(e, w, bm=4000):
    return pl.pallas_call(
        _eupd_body,
        grid=(E // bm,),
        in_specs=[
            pl.BlockSpec((bm, D), lambda i: (i, 0)),
            pl.BlockSpec((D, D), lambda i: (0, 0)),
        ],
        out_specs=pl.BlockSpec((bm, D), lambda i: (i, 0)),
        out_shape=jax.ShapeDtypeStruct((E, D), jnp.float32),
    )(e, w)


def _hupd_body(agg_ref, w_ref, r_ref, o_ref):
    o_ref[...] = jnp.maximum(
        jnp.dot(agg_ref[...], w_ref[...], preferred_element_type=jnp.float32),
        0.0) + r_ref[...]


def _h_update(agg, w, res, bm=2000):
    return pl.pallas_call(
        _hupd_body,
        grid=(N // bm,),
        in_specs=[
            pl.BlockSpec((bm, D), lambda i: (i, 0)),
            pl.BlockSpec((D, D), lambda i: (0, 0)),
            pl.BlockSpec((bm, D), lambda i: (i, 0)),
        ],
        out_specs=pl.BlockSpec((bm, D), lambda i: (i, 0)),
        out_shape=jax.ShapeDtypeStruct((N, D), jnp.float32),
    )(agg, w, res)


def _add2_body(a_ref, b_ref, o_ref, *, scale):
    o_ref[...] = a_ref[...] + b_ref[...] * scale


def _add2(a, b, scale, bm=2000):
    return pl.pallas_call(
        functools.partial(_add2_body, scale=scale),
        grid=(N // bm,),
        in_specs=[
            pl.BlockSpec((bm, D), lambda i: (i, 0)),
            pl.BlockSpec((bm, D), lambda i: (i, 0)),
        ],
        out_specs=pl.BlockSpec((bm, D), lambda i: (i, 0)),
        out_shape=jax.ShapeDtypeStruct((N, D), jnp.float32),
    )(a, b)


def _agg(h, e, src, dst):
    m = jax.nn.relu(h[src] + e)
    return jax.ops.segment_sum(m, dst, num_segments=N)


def kernel(x, edge_index, edge_attr, W_atom, W_bond, Wn, We, node_rand, edge_rand):
    src = edge_index[0]
    dst = edge_index[1]
    node_mask = (node_rand > P_NODE).astype(jnp.float32)[:, None]
    edge_mask = (edge_rand > P_EDGE).astype(jnp.float32)[:, None]

    h0 = _embed(x, W_atom, node_mask, bm=2000)
    e0 = _embed(edge_attr, W_bond, edge_mask, bm=4000)

    def mp(h, e, i):
        agg = _agg(h, e, src, dst)
        return _h_update(agg, Wn[i], h), _e_update(e, We[i])

    h1, e1 = mp(h0, e0, 0)
    h2, e2 = mp(h1, e1, 1)
    g = _add2(h2, h1, 1.0)
    u00h, u00e = mp(g, e2, 3)
    xs0 = _add2(h1, u00h, 1.0)
    h3, e3 = mp(h2, e2, 2)
    g = _add2(h3, h2, 1.0)
    u11h, u11e = mp(g, e3, 5)
    g = _add2(u11h, xs0, 0.5)
    u10h, u10e = mp(g, u11e, 4)

    return (jnp.stack([u00h, u10h]), jnp.stack([u00e, u10e]),
            node_mask.reshape(-1), edge_mask.reshape(-1))


# speedup vs baseline: 2.7139x; 2.6327x over previous
import functools

import jax
import jax.numpy as jnp
from jax import lax
from jax.experimental import pallas as pl
from jax.experimental.pallas import tpu as pltpu
from jax.experimental.pallas import tpu_sc as plsc

N = 10000
E = 320000
DIN = 128
DE = 16
D = 128
P_NODE = 0.1
P_EDGE = 0.1

NPAD = 10240
CHUNK = 128
NCHUNKS = E // CHUNK
NW = 32


def _embed_body(a_ref, w_ref, m_ref, o_ref):
    o_ref[...] = jnp.dot(a_ref[...], w_ref[...],
                         preferred_element_type=jnp.float32) * m_ref[...]


def _embed(a, w, m, bm):
    r, k = a.shape
    return pl.pallas_call(
        _embed_body,
        grid=(r // bm,),
        in_specs=[
            pl.BlockSpec((bm, k), lambda i: (i, 0)),
            pl.BlockSpec((k, D), lambda i: (0, 0)),
            pl.BlockSpec((bm, 1), lambda i: (i, 0)),
        ],
        out_specs=pl.BlockSpec((bm, D), lambda i: (i, 0)),
        out_shape=jax.ShapeDtypeStruct((r, D), jnp.float32),
    )(a, w, m)


def _eupd_body(e_ref, w_ref, o_ref):
    e = e_ref[...]
    o_ref[...] = jnp.maximum(
        jnp.dot(e, w_ref[...], preferred_element_type=jnp.float32), 0.0) + e


def _e_update(e, w, bm=4000):
    return pl.pallas_call(
        _eupd_body,
        grid=(E // bm,),
        in_specs=[
            pl.BlockSpec((bm, D), lambda i: (i, 0)),
            pl.BlockSpec((D, D), lambda i: (0, 0)),
        ],
        out_specs=pl.BlockSpec((bm, D), lambda i: (i, 0)),
        out_shape=jax.ShapeDtypeStruct((E, D), jnp.float32),
    )(e, w)


def _hupd_body(p_ref, w_ref, r_ref, o_ref):
    agg = p_ref[0] + p_ref[1]
    o_ref[...] = jnp.maximum(
        jnp.dot(agg, w_ref[...], preferred_element_type=jnp.float32),
        0.0) + r_ref[...]


def _h_update(parts, w, res, bm=2000):
    return pl.pallas_call(
        _hupd_body,
        grid=(N // bm,),
        in_specs=[
            pl.BlockSpec((2, bm, D), lambda i: (0, i, 0)),
            pl.BlockSpec((D, D), lambda i: (0, 0)),
            pl.BlockSpec((bm, D), lambda i: (i, 0)),
        ],
        out_specs=pl.BlockSpec((bm, D), lambda i: (i, 0)),
        out_shape=jax.ShapeDtypeStruct((N, D), jnp.float32),
    )(parts, w, res)


def _add2_body(a_ref, b_ref, o_ref, *, scale):
    o_ref[...] = a_ref[...] + b_ref[...] * scale


def _add2(a, b, scale, bm=2000):
    return pl.pallas_call(
        functools.partial(_add2_body, scale=scale),
        grid=(N // bm,),
        in_specs=[
            pl.BlockSpec((bm, D), lambda i: (i, 0)),
            pl.BlockSpec((bm, D), lambda i: (i, 0)),
        ],
        out_specs=pl.BlockSpec((bm, D), lambda i: (i, 0)),
        out_shape=jax.ShapeDtypeStruct((N, D), jnp.float32),
    )(a, b)


def _sc_agg_body(h_hbm, e_hbm, src_hbm, dst_hbm, out_hbm,
                 e_buf, h_buf, sidx, didx, acc, sem):
    c = lax.axis_index("c")
    s = lax.axis_index("s")
    w = s * 2 + c

    def zrow(r, _):
        for k in range(D // 16):
            e_buf[r, pl.ds(k * 16, 16)] = jnp.zeros((16,), jnp.float32)
        return 0
    lax.fori_loop(0, CHUNK, zrow, 0)
    for z in range(NPAD // 16 // CHUNK):
        pltpu.sync_copy(e_buf, acc.at[pl.ds(s * (NPAD // 16) + z * CHUNK, CHUNK)])
    plsc.subcore_barrier()

    base = w * (NCHUNKS // NW) + jnp.minimum(w, NCHUNKS % NW)
    nch = (NCHUNKS // NW) + jnp.where(w < NCHUNKS % NW, 1, 0)

    def chunk_body(t, _):
        cid = base + t
        pltpu.sync_copy(e_hbm.at[pl.ds(cid * CHUNK, CHUNK)], e_buf)
        pltpu.sync_copy(src_hbm.at[cid], sidx)
        pltpu.sync_copy(dst_hbm.at[cid], didx)
        pltpu.async_copy(h_hbm.at[sidx], h_buf, sem).wait()

        def mrow(r, _):
            for k in range(D // 16):
                sl = pl.ds(k * 16, 16)
                e_buf[r, sl] = jnp.maximum(e_buf[r, sl] + h_buf[r, sl], 0.0)
            return 0
        lax.fori_loop(0, CHUNK, mrow, 0)
        pltpu.sync_copy(e_buf, acc.at[didx], add=True)
        return 0
    lax.fori_loop(0, nch, chunk_body, 0)
    plsc.subcore_barrier()

    for z in range(NPAD // 16 // CHUNK):
        rows = pl.ds(s * (NPAD // 16) + z * CHUNK, CHUNK)
        pltpu.sync_copy(acc.at[rows], out_hbm.at[c].at[rows])


@functools.partial(jax.jit, static_argnames=())
def _sc_agg(h, e, src2d, dst2d):
    mesh = plsc.VectorSubcoreMesh(core_axis_name="c", subcore_axis_name="s")
    f = pl.kernel(
        _sc_agg_body,
        mesh=mesh,
        out_type=jax.ShapeDtypeStruct((2, NPAD, D), jnp.float32),
        scratch_types=[
            pltpu.VMEM((CHUNK, D), jnp.float32),
            pltpu.VMEM((CHUNK, D), jnp.float32),
            pltpu.VMEM((CHUNK,), jnp.int32),
            pltpu.VMEM((CHUNK,), jnp.int32),
            pltpu.VMEM_SHARED((NPAD, D), jnp.float32),
            pltpu.SemaphoreType.DMA,
        ],
    )
    return f(h, e, src2d, dst2d)


def kernel(x, edge_index, edge_attr, W_atom, W_bond, Wn, We, node_rand, edge_rand):
    src2d = edge_index[0].reshape(NCHUNKS, CHUNK)
    dst2d = edge_index[1].reshape(NCHUNKS, CHUNK)
    node_mask = (node_rand > P_NODE).astype(jnp.float32)[:, None]
    edge_mask = (edge_rand > P_EDGE).astype(jnp.float32)[:, None]

    h0 = _embed(x, W_atom, node_mask, bm=2000)
    e0 = _embed(edge_attr, W_bond, edge_mask, bm=4000)

    def mp(h, e, i):
        parts = _sc_agg(h, e, src2d, dst2d)
        return _h_update(parts, Wn[i], h), _e_update(e, We[i])

    h1, e1 = mp(h0, e0, 0)
    h2, e2 = mp(h1, e1, 1)
    g = _add2(h2, h1, 1.0)
    u00h, u00e = mp(g, e2, 3)
    xs0 = _add2(h1, u00h, 1.0)
    h3, e3 = mp(h2, e2, 2)
    g = _add2(h3, h2, 1.0)
    u11h, u11e = mp(g, e3, 5)
    g = _add2(u11h, xs0, 0.5)
    u10h, u10e = mp(g, u11e, 4)

    return (jnp.stack([u00h, u10h]), jnp.stack([u00e, u10e]),
            node_mask.reshape(-1), edge_mask.reshape(-1))
